# Initial kernel scaffold; baseline (speedup 1.0000x reference)
import jax
import jax.numpy as jnp
from jax.experimental import pallas as pl


def kernel(x, d, index, wx0, bx0, wx1, bx1, wx2, bx2, wx3, bx3, wx4, bx4, wx5, bx5, wx6, bx6, wx7, bx7, wint, bint, wden, bden, wc1, bc1, wc2, bc2):
    raise NotImplementedError("write your pallas kernel here")


# speedup vs baseline: 2.1629x; 2.1629x over previous
import functools

import jax
import jax.numpy as jnp
import numpy as np
from jax.experimental import pallas as pl
from jax.experimental.pallas import tpu as pltpu

E = 100
HX = 128
HD = 64
NHX = 6
NHD = 4
B = 4096
DIMX = 3 * NHX * 2
DIMD = 3 * NHD * 2

T = 32
NT = B // T + E
NS = NT * T


def _harm_tile(v, n):
    f = jnp.exp2(jax.lax.broadcasted_iota(jnp.int32, (1, n), 1).astype(jnp.float32))
    cols = [v[:, i:i + 1] * f for i in range(3)]
    e = jnp.concatenate(cols, axis=-1)
    return jnp.concatenate([jnp.sin(e), jnp.cos(e)], axis=-1)


def _moe_body(te_ref, xg_ref, dg_ref,
              wx0, bx0, wx1, bx1, wx2, bx2, wx3, bx3,
              wx4, bx4, wx5, bx5, wx6, bx6, wx7, bx7,
              wint, bint, wden, bden, wc1, bc1, wc2, bc2,
              out_ref):
    ws = [wx0, wx1, wx2, wx3, wx4, wx5, wx6, wx7]
    bs = [bx0, bx1, bx2, bx3, bx4, bx5, bx6, bx7]
    ex = _harm_tile(xg_ref[:], NHX)
    ed = _harm_tile(dg_ref[:], NHD)
    y = ex
    for li in range(8):
        if li == 5:
            y = jnp.concatenate([y, ex], axis=-1)
        h = jnp.dot(y, ws[li][0], preferred_element_type=jnp.float32)
        y = jnp.maximum(h + bs[li][0], 0.0)
    density = jnp.dot(y, wden[0], preferred_element_type=jnp.float32) + bden[0]
    inter = jnp.dot(y, wint[0], preferred_element_type=jnp.float32) + bint[0]
    ci = jnp.concatenate([inter, ed], axis=-1)
    c = jnp.maximum(
        jnp.dot(ci, wc1[0], preferred_element_type=jnp.float32) + bc1[0], 0.0)
    color = jax.nn.sigmoid(
        jnp.dot(c, wc2[0], preferred_element_type=jnp.float32) + bc2[0])
    out_ref[:] = jnp.concatenate([density, color], axis=-1)


def _wspec(din, dout):
    return pl.BlockSpec((1, din, dout), lambda t, te: (te[t], 0, 0))


def _bspec(dout):
    return pl.BlockSpec((1, 1, dout), lambda t, te: (te[t], 0, 0))


@jax.jit
def kernel(x, d, index, wx0, bx0, wx1, bx1, wx2, bx2, wx3, bx3, wx4, bx4,
           wx5, bx5, wx6, bx6, wx7, bx7, wint, bint, wden, bden, wc1, bc1,
           wc2, bc2):
    bx0, bx1, bx2, bx3, bx4, bx5, bx6, bx7, bint, bden, bc1, bc2 = (
        b.reshape(E, 1, -1) for b in
        (bx0, bx1, bx2, bx3, bx4, bx5, bx6, bx7, bint, bden, bc1, bc2))
    idx = index.astype(jnp.int32)
    order = jnp.argsort(idx).astype(jnp.int32)
    counts = jnp.bincount(idx, length=E).astype(jnp.int32)
    starts = jnp.concatenate(
        [jnp.zeros((1,), jnp.int32), jnp.cumsum(counts)[:-1].astype(jnp.int32)])
    pad_counts = ((counts + T - 1) // T) * T
    pcsum = jnp.cumsum(pad_counts).astype(jnp.int32)
    pad_starts = pcsum - pad_counts

    tile_e = jnp.searchsorted(
        pcsum, jnp.arange(NT, dtype=jnp.int32) * T, side='right').astype(jnp.int32)
    tile_e = jnp.minimum(tile_e, E - 1)

    s = jnp.arange(NS, dtype=jnp.int32)
    se = jnp.repeat(tile_e, T)
    p = s - pad_starts[se]
    j = starts[se] + jnp.minimum(p, jnp.maximum(counts[se] - 1, 0))
    src = order[jnp.clip(j, 0, B - 1)]
    xg = x[src]
    dg = d[src]

    grid_spec = pltpu.PrefetchScalarGridSpec(
        num_scalar_prefetch=1,
        grid=(NT,),
        in_specs=[
            pl.BlockSpec((T, 3), lambda t, te: (t, 0)),
            pl.BlockSpec((T, 3), lambda t, te: (t, 0)),
            _wspec(DIMX, HX), _bspec(HX),
            _wspec(HX, HX), _bspec(HX),
            _wspec(HX, HX), _bspec(HX),
            _wspec(HX, HX), _bspec(HX),
            _wspec(HX, HX), _bspec(HX),
            _wspec(HX + DIMX, HX), _bspec(HX),
            _wspec(HX, HX), _bspec(HX),
            _wspec(HX, HX), _bspec(HX),
            _wspec(HX, HX), _bspec(HX),
            _wspec(HX, 1), _bspec(1),
            _wspec(HX + DIMD, HD), _bspec(HD),
            _wspec(HD, 3), _bspec(3),
        ],
        out_specs=pl.BlockSpec((T, 4), lambda t, te: (t, 0)),
    )
    outp = pl.pallas_call(
        _moe_body,
        grid_spec=grid_spec,
        out_shape=jax.ShapeDtypeStruct((NS, 4), jnp.float32),
    )(tile_e, xg, dg,
      wx0, bx0, wx1, bx1, wx2, bx2, wx3, bx3, wx4, bx4, wx5, bx5,
      wx6, bx6, wx7, bx7, wint, bint, wden, bden, wc1, bc1, wc2, bc2)

    inv = jnp.zeros((B,), jnp.int32).at[order].set(jnp.arange(B, dtype=jnp.int32))
    slot_of_row = pad_starts[idx] + (inv - starts[idx])
    return outp[slot_of_row]
